# Initial kernel scaffold; baseline (speedup 1.0000x reference)
import jax
import jax.numpy as jnp
from jax.experimental import pallas as pl


def kernel(x, pos, edge_index, batch, node_in_W, node_in_b, edge_W1, edge_b1, edge_W2, edge_b2, conv_W1, conv_b1, conv_W2, conv_b2, head_W1, head_b1, head_W2, head_b2):
    raise NotImplementedError("write your pallas kernel here")


# speedup vs baseline: 2.3466x; 2.3466x over previous
import jax
import jax.numpy as jnp
from jax import lax
from jax.experimental import pallas as pl
from jax.experimental.pallas import tpu as pltpu
from jax.experimental.pallas import tpu_sc as plsc

N = 10000
E = 320000
H = 256
HH = 128
B = 64
L = 5
RBF_K = 32

NC = 2
NS = 16
LANES = 16

NP = 10240
ROWS_PT = NP // NS
CH = 128
CHUNKS = 158
EPC = CH * CHUNKS
EP = NS * EPC
EPT32 = EP // (NC * NS)
EB = 2048
NB = 1280

GAMMA = 1.0 / (2.0 * (5.0 / RBF_K) ** 2)
CSTEP = 5.0 / (RBF_K - 1)


def _sc_dist_body(px, py, pz, src, dst, out, pxv, pyv, pzv, sv, dv, d2v):
    wid = lax.axis_index("s") * NC + lax.axis_index("c")
    base = wid * EPT32
    pltpu.sync_copy(px, pxv)
    pltpu.sync_copy(py, pyv)
    pltpu.sync_copy(pz, pzv)
    pltpu.sync_copy(src.at[pl.ds(base, EPT32)], sv)
    pltpu.sync_copy(dst.at[pl.ds(base, EPT32)], dv)

    def body(g, _):
        sl = pl.ds(g * LANES, LANES)
        s16 = sv[sl]
        d16 = dv[sl]
        dx = plsc.load_gather(pxv, [s16]) - plsc.load_gather(pxv, [d16])
        dy = plsc.load_gather(pyv, [s16]) - plsc.load_gather(pyv, [d16])
        dz = plsc.load_gather(pzv, [s16]) - plsc.load_gather(pzv, [d16])
        d2v[sl] = dx * dx + dy * dy + dz * dz
        return _

    lax.fori_loop(0, EPT32 // LANES, body, None)
    pltpu.sync_copy(d2v, out.at[pl.ds(base, EPT32)])


def _sc_dist(px, py, pz, src, dst):
    return pl.kernel(
        _sc_dist_body,
        out_type=jax.ShapeDtypeStruct((EP,), jnp.float32),
        mesh=plsc.VectorSubcoreMesh(core_axis_name="c", subcore_axis_name="s"),
        scratch_types=[
            pltpu.VMEM((N,), jnp.float32),
            pltpu.VMEM((N,), jnp.float32),
            pltpu.VMEM((N,), jnp.float32),
            pltpu.VMEM((EPT32,), jnp.int32),
            pltpu.VMEM((EPT32,), jnp.int32),
            pltpu.VMEM((EPT32,), jnp.float32),
        ],
        compiler_params=pltpu.CompilerParams(needs_layout_passes=False),
    )(px, py, pz, src, dst)


def _sc_msg_half(h_ref, e_ref, src, dst, out_ref, acc, idx_s, idx_d, hbuf, ebuf):
    tid = lax.axis_index("s")
    tbase = tid * EPC
    rbase = tid * ROWS_PT

    zero16 = jnp.zeros((LANES,), jnp.float32)

    def zbody(r, _):
        for k in range(HH // LANES):
            ebuf[r, pl.ds(k * LANES, LANES)] = zero16
        return _

    lax.fori_loop(0, CH, zbody, None)
    for j in range(ROWS_PT // CH):
        pltpu.sync_copy(ebuf, acc.at[pl.ds(rbase + j * CH, CH)])
    plsc.subcore_barrier()

    def chunk(c, _):
        base = tbase + c * CH
        pltpu.sync_copy(src.at[pl.ds(base, CH)], idx_s.at[0])
        pltpu.sync_copy(dst.at[pl.ds(base, CH)], idx_d.at[0])
        pltpu.sync_copy(h_ref.at[idx_s.at[0]], hbuf)
        pltpu.sync_copy(e_ref.at[pl.ds(base, CH)], ebuf)

        def rbody(r, _):
            for k in range(HH // LANES):
                sl = pl.ds(k * LANES, LANES)
                hbuf[r, sl] = jnp.maximum(hbuf[r, sl] + ebuf[r, sl], 0.0)
            return _

        lax.fori_loop(0, CH, rbody, None)
        pltpu.sync_copy(hbuf, acc.at[idx_d.at[0]], add=True)
        return _

    lax.fori_loop(0, CHUNKS, chunk, None)
    plsc.subcore_barrier()

    for j in range(ROWS_PT // CH):
        sl = pl.ds(rbase + j * CH, CH)
        pltpu.sync_copy(acc.at[sl], hbuf)
        pltpu.sync_copy(hbuf, out_ref.at[sl])


def _sc_msg_body(h_lo, h_hi, e_lo, e_hi, src, dst, agg_lo, agg_hi,
                 acc, idx_s, idx_d, hbuf, ebuf):
    c = lax.axis_index("c")

    @pl.when(c == 0)
    def _():
        _sc_msg_half(h_lo, e_lo, src, dst, agg_lo, acc, idx_s, idx_d, hbuf, ebuf)

    @pl.when(c == 1)
    def _():
        _sc_msg_half(h_hi, e_hi, src, dst, agg_hi, acc, idx_s, idx_d, hbuf, ebuf)


def _sc_msg(h_lo, h_hi, e_lo, e_hi, src, dst):
    return pl.kernel(
        _sc_msg_body,
        out_type=[
            jax.ShapeDtypeStruct((NP, HH), jnp.float32),
            jax.ShapeDtypeStruct((NP, HH), jnp.float32),
        ],
        mesh=plsc.VectorSubcoreMesh(core_axis_name="c", subcore_axis_name="s"),
        scratch_types=[
            pltpu.VMEM_SHARED((NP, HH), jnp.float32),
            pltpu.VMEM((1, CH), jnp.int32),
            pltpu.VMEM((1, CH), jnp.int32),
            pltpu.VMEM((CH, HH), jnp.float32),
            pltpu.VMEM((CH, HH), jnp.float32),
        ],
        compiler_params=pltpu.CompilerParams(needs_layout_passes=False),
    )(h_lo, h_hi, e_lo, e_hi, src, dst)


def _edge_mlp_kernel(d2_ref, w1_ref, b1_ref, w2_ref, b2_ref, elo_ref, ehi_ref):
    d = jnp.sqrt(jnp.maximum(d2_ref[...], 0.0))
    centers = lax.broadcasted_iota(jnp.int32, (EB, RBF_K), 1).astype(jnp.float32) * CSTEP
    rbf = jnp.exp(-GAMMA * (d - centers) ** 2)
    t = jnp.dot(rbf, w1_ref[...], preferred_element_type=jnp.float32)
    t = jnp.maximum(t + b1_ref[...], 0.0)
    e = jnp.dot(t, w2_ref[...], preferred_element_type=jnp.float32) + b2_ref[...]
    elo_ref[...] = e[:, :HH]
    ehi_ref[...] = e[:, HH:]


def _tc_edge_mlp(d2, w1, b1, w2, b2):
    return pl.pallas_call(
        _edge_mlp_kernel,
        grid=(EP // EB,),
        in_specs=[
            pl.BlockSpec((EB, 1), lambda i: (i, 0)),
            pl.BlockSpec((RBF_K, H), lambda i: (0, 0)),
            pl.BlockSpec((1, H), lambda i: (0, 0)),
            pl.BlockSpec((H, H), lambda i: (0, 0)),
            pl.BlockSpec((1, H), lambda i: (0, 0)),
        ],
        out_specs=[
            pl.BlockSpec((EB, HH), lambda i: (i, 0)),
            pl.BlockSpec((EB, HH), lambda i: (i, 0)),
        ],
        out_shape=[
            jax.ShapeDtypeStruct((EP, HH), jnp.float32),
            jax.ShapeDtypeStruct((EP, HH), jnp.float32),
        ],
    )(d2, w1, b1, w2, b2)


def _h0_kernel(x_ref, w_ref, b_ref, lo_ref, hi_ref):
    h = jnp.dot(x_ref[...], w_ref[...], preferred_element_type=jnp.float32)
    h = h + b_ref[...]
    lo_ref[...] = h[:, :HH]
    hi_ref[...] = h[:, HH:]


def _tc_h0(xp, w, b):
    return pl.pallas_call(
        _h0_kernel,
        grid=(NP // NB,),
        in_specs=[
            pl.BlockSpec((NB, 16), lambda i: (i, 0)),
            pl.BlockSpec((16, H), lambda i: (0, 0)),
            pl.BlockSpec((1, H), lambda i: (0, 0)),
        ],
        out_specs=[
            pl.BlockSpec((NB, HH), lambda i: (i, 0)),
            pl.BlockSpec((NB, HH), lambda i: (i, 0)),
        ],
        out_shape=[
            jax.ShapeDtypeStruct((NP, HH), jnp.float32),
            jax.ShapeDtypeStruct((NP, HH), jnp.float32),
        ],
    )(xp, w, b)


def _node_mlp_kernel(hlo_ref, hhi_ref, alo_ref, ahi_ref,
                     w1_ref, b1_ref, w2_ref, b2_ref, olo_ref, ohi_ref):
    zlo = hlo_ref[...] + alo_ref[...]
    zhi = hhi_ref[...] + ahi_ref[...]
    w1 = w1_ref[...]
    t = jnp.dot(zlo, w1[:HH, :], preferred_element_type=jnp.float32)
    t = t + jnp.dot(zhi, w1[HH:, :], preferred_element_type=jnp.float32)
    t = jnp.maximum(t + b1_ref[...], 0.0)
    u = jnp.dot(t, w2_ref[...], preferred_element_type=jnp.float32) + b2_ref[...]
    olo_ref[...] = jnp.maximum(u[:, :HH], 0.0) + hlo_ref[...]
    ohi_ref[...] = jnp.maximum(u[:, HH:], 0.0) + hhi_ref[...]


def _tc_node_mlp(h_lo, h_hi, a_lo, a_hi, w1, b1, w2, b2):
    return pl.pallas_call(
        _node_mlp_kernel,
        grid=(NP // NB,),
        in_specs=[
            pl.BlockSpec((NB, HH), lambda i: (i, 0)),
            pl.BlockSpec((NB, HH), lambda i: (i, 0)),
            pl.BlockSpec((NB, HH), lambda i: (i, 0)),
            pl.BlockSpec((NB, HH), lambda i: (i, 0)),
            pl.BlockSpec((H, H), lambda i: (0, 0)),
            pl.BlockSpec((1, H), lambda i: (0, 0)),
            pl.BlockSpec((H, H), lambda i: (0, 0)),
            pl.BlockSpec((1, H), lambda i: (0, 0)),
        ],
        out_specs=[
            pl.BlockSpec((NB, HH), lambda i: (i, 0)),
            pl.BlockSpec((NB, HH), lambda i: (i, 0)),
        ],
        out_shape=[
            jax.ShapeDtypeStruct((NP, HH), jnp.float32),
            jax.ShapeDtypeStruct((NP, HH), jnp.float32),
        ],
    )(h_lo, h_hi, a_lo, a_hi, w1, b1, w2, b2)


def _pool_head_kernel(b_ref, hlo_ref, hhi_ref, w1_ref, b1_ref, w2_ref, b2_ref,
                      out_ref, glo, ghi, cnt):
    i = pl.program_id(0)

    @pl.when(i == 0)
    def _():
        glo[...] = jnp.zeros_like(glo)
        ghi[...] = jnp.zeros_like(ghi)
        cnt[...] = jnp.zeros_like(cnt)

    seg = b_ref[0]
    gids = lax.broadcasted_iota(jnp.int32, (B, NB), 0)
    m = (seg == gids).astype(jnp.float32)
    glo[...] += jnp.dot(m, hlo_ref[...], preferred_element_type=jnp.float32)
    ghi[...] += jnp.dot(m, hhi_ref[...], preferred_element_type=jnp.float32)
    cnt[...] += jnp.sum(m, axis=1, keepdims=True)

    @pl.when(i == pl.num_programs(0) - 1)
    def _():
        c = jnp.maximum(cnt[...], 1.0)
        gl = glo[...] / c
        gh = ghi[...] / c
        w1 = w1_ref[...]
        t = jnp.dot(gl, w1[:HH, :], preferred_element_type=jnp.float32)
        t = t + jnp.dot(gh, w1[HH:, :], preferred_element_type=jnp.float32)
        t = jnp.maximum(t + b1_ref[...], 0.0)
        out_ref[...] = (jnp.dot(t, w2_ref[...], preferred_element_type=jnp.float32)
                        + b2_ref[...])


def _tc_pool_head(batch2, h_lo, h_hi, w1, b1, w2, b2):
    return pl.pallas_call(
        _pool_head_kernel,
        grid=(NP // NB,),
        in_specs=[
            pl.BlockSpec((1, 1, NB), lambda i: (i, 0, 0)),
            pl.BlockSpec((NB, HH), lambda i: (i, 0)),
            pl.BlockSpec((NB, HH), lambda i: (i, 0)),
            pl.BlockSpec((H, HH), lambda i: (0, 0)),
            pl.BlockSpec((1, HH), lambda i: (0, 0)),
            pl.BlockSpec((HH, 1), lambda i: (0, 0)),
            pl.BlockSpec((1, 1), lambda i: (0, 0)),
        ],
        out_specs=pl.BlockSpec((B, 1), lambda i: (0, 0)),
        out_shape=jax.ShapeDtypeStruct((B, 1), jnp.float32),
        scratch_shapes=[
            pltpu.VMEM((B, HH), jnp.float32),
            pltpu.VMEM((B, HH), jnp.float32),
            pltpu.VMEM((B, 1), jnp.float32),
        ],
    )(batch2, h_lo, h_hi, w1, b1, w2, b2)


def kernel(x, pos, edge_index, batch, node_in_W, node_in_b, edge_W1, edge_b1,
           edge_W2, edge_b2, conv_W1, conv_b1, conv_W2, conv_b2,
           head_W1, head_b1, head_W2, head_b2):
    f32 = jnp.float32
    px = pos[:, 0]
    py = pos[:, 1]
    pz = pos[:, 2]
    src = edge_index[0]
    dst = edge_index[1]
    src_p = jnp.concatenate([src, jnp.zeros((EP - E,), jnp.int32)])
    dst_p = jnp.concatenate([dst, jnp.full((EP - E,), N, jnp.int32)])
    xp = jnp.zeros((NP, 16), f32).at[:N, :9].set(x)
    w16 = jnp.zeros((16, H), f32).at[:9, :].set(node_in_W)
    batch_p = jnp.concatenate([batch, jnp.full((NP - N,), B, jnp.int32)])
    batch2 = batch_p.reshape(NP // NB, 1, NB)

    d2 = _sc_dist(px, py, pz, src_p, dst_p).reshape(EP, 1)
    e_lo, e_hi = _tc_edge_mlp(d2, edge_W1, edge_b1.reshape(1, H),
                              edge_W2, edge_b2.reshape(1, H))

    h_lo, h_hi = _tc_h0(xp, w16, node_in_b.reshape(1, H))

    for i in range(L):
        a_lo, a_hi = _sc_msg(h_lo, h_hi, e_lo, e_hi, src_p, dst_p)
        h_lo, h_hi = _tc_node_mlp(h_lo, h_hi, a_lo, a_hi,
                                  conv_W1[i], conv_b1[i].reshape(1, H),
                                  conv_W2[i], conv_b2[i].reshape(1, H))

    out = _tc_pool_head(batch2, h_lo, h_hi, head_W1, head_b1.reshape(1, HH),
                        head_W2, head_b2.reshape(1, 1))
    return out[:, 0]


# speedup vs baseline: 3.2988x; 1.4058x over previous
import jax
import jax.numpy as jnp
from jax import lax
from jax.experimental import pallas as pl
from jax.experimental.pallas import tpu as pltpu
from jax.experimental.pallas import tpu_sc as plsc

N = 10000
E = 320000
H = 256
HH = 128
B = 64
L = 5
RBF_K = 32

NC = 2
NS = 16
LANES = 16

NP = 10240
ROWS_PT = NP // NS
CH = 64
CHUNKS = 316
EPC = CH * CHUNKS
EP = NS * EPC
EPT32 = EP // (NC * NS)
EB = 2048
NB = 1280

GAMMA = 1.0 / (2.0 * (5.0 / RBF_K) ** 2)
CSTEP = 5.0 / (RBF_K - 1)


def _sc_dist_body(px, py, pz, src, dst, out, pxv, pyv, pzv, sv, dv, d2v):
    wid = lax.axis_index("s") * NC + lax.axis_index("c")
    base = wid * EPT32
    pltpu.sync_copy(px, pxv)
    pltpu.sync_copy(py, pyv)
    pltpu.sync_copy(pz, pzv)
    pltpu.sync_copy(src.at[pl.ds(base, EPT32)], sv)
    pltpu.sync_copy(dst.at[pl.ds(base, EPT32)], dv)

    def body(g, _):
        sl = pl.ds(g * LANES, LANES)
        s16 = sv[sl]
        d16 = dv[sl]
        dx = plsc.load_gather(pxv, [s16]) - plsc.load_gather(pxv, [d16])
        dy = plsc.load_gather(pyv, [s16]) - plsc.load_gather(pyv, [d16])
        dz = plsc.load_gather(pzv, [s16]) - plsc.load_gather(pzv, [d16])
        d2v[sl] = dx * dx + dy * dy + dz * dz
        return _

    lax.fori_loop(0, EPT32 // LANES, body, None)
    pltpu.sync_copy(d2v, out.at[pl.ds(base, EPT32)])


def _sc_dist(px, py, pz, src, dst):
    return pl.kernel(
        _sc_dist_body,
        out_type=jax.ShapeDtypeStruct((EP,), jnp.float32),
        mesh=plsc.VectorSubcoreMesh(core_axis_name="c", subcore_axis_name="s"),
        scratch_types=[
            pltpu.VMEM((N,), jnp.float32),
            pltpu.VMEM((N,), jnp.float32),
            pltpu.VMEM((N,), jnp.float32),
            pltpu.VMEM((EPT32,), jnp.int32),
            pltpu.VMEM((EPT32,), jnp.int32),
            pltpu.VMEM((EPT32,), jnp.float32),
        ],
        compiler_params=pltpu.CompilerParams(needs_layout_passes=False),
    )(px, py, pz, src, dst)


def _sc_msg_half(h_ref, e_ref, src, dst, out_ref, acc, idx_s, idx_d, hbuf, ebuf,
                 sem0, sem1):
    tid = lax.axis_index("s")
    tbase = tid * EPC
    rbase = tid * ROWS_PT
    sems = (sem0, sem1)

    zero16 = jnp.zeros((LANES,), jnp.float32)

    def zbody(r, _):
        for k in range(HH // LANES):
            ebuf[0, r, pl.ds(k * LANES, LANES)] = zero16
        return _

    lax.fori_loop(0, CH, zbody, None)
    for j in range(ROWS_PT // CH):
        pltpu.sync_copy(ebuf.at[0], acc.at[pl.ds(rbase + j * CH, CH)])
    plsc.subcore_barrier()

    def issue(c, b):
        base = tbase + c * CH
        pltpu.sync_copy(src.at[pl.ds(base, CH)], idx_s.at[b])
        pltpu.sync_copy(dst.at[pl.ds(base, CH)], idx_d.at[b])
        pltpu.async_copy(h_ref.at[idx_s.at[b]], hbuf.at[b], sems[b])
        pltpu.async_copy(e_ref.at[pl.ds(base, CH)], ebuf.at[b], sems[b])

    for b in range(2):
        issue(b, b)

    @pl.loop(0, CHUNKS, step=2)
    def _(j):
        for b in range(2):
            pltpu.make_async_copy(h_ref.at[idx_s.at[b]], hbuf.at[b],
                                  sems[b]).wait()
            pltpu.make_async_copy(e_ref.at[pl.ds(0, CH)], ebuf.at[b],
                                  sems[b]).wait()

            def rbody(r, _):
                for k in range(HH // LANES):
                    sl = pl.ds(k * LANES, LANES)
                    hbuf[b, r, sl] = jnp.maximum(hbuf[b, r, sl] + ebuf[b, r, sl],
                                                 0.0)
                return _

            lax.fori_loop(0, CH, rbody, None)
            pltpu.sync_copy(hbuf.at[b], acc.at[idx_d.at[b]], add=True)

            @pl.when(j + b + 2 < CHUNKS)
            def _():
                issue(j + b + 2, b)

    plsc.subcore_barrier()

    for j in range(ROWS_PT // CH):
        sl = pl.ds(rbase + j * CH, CH)
        pltpu.sync_copy(acc.at[sl], hbuf.at[0])
        pltpu.sync_copy(hbuf.at[0], out_ref.at[sl])


def _sc_msg_body(h_lo, h_hi, e_lo, e_hi, src, dst, agg_lo, agg_hi,
                 acc, idx_s, idx_d, hbuf, ebuf, sem0, sem1):
    c = lax.axis_index("c")

    @pl.when(c == 0)
    def _():
        _sc_msg_half(h_lo, e_lo, src, dst, agg_lo, acc, idx_s, idx_d,
                     hbuf, ebuf, sem0, sem1)

    @pl.when(c == 1)
    def _():
        _sc_msg_half(h_hi, e_hi, src, dst, agg_hi, acc, idx_s, idx_d,
                     hbuf, ebuf, sem0, sem1)


def _sc_msg(h_lo, h_hi, e_lo, e_hi, src, dst):
    return pl.kernel(
        _sc_msg_body,
        out_type=[
            jax.ShapeDtypeStruct((NP, HH), jnp.float32),
            jax.ShapeDtypeStruct((NP, HH), jnp.float32),
        ],
        mesh=plsc.VectorSubcoreMesh(core_axis_name="c", subcore_axis_name="s"),
        scratch_types=[
            pltpu.VMEM_SHARED((NP, HH), jnp.float32),
            pltpu.VMEM((2, CH), jnp.int32),
            pltpu.VMEM((2, CH), jnp.int32),
            pltpu.VMEM((2, CH, HH), jnp.float32),
            pltpu.VMEM((2, CH, HH), jnp.float32),
            pltpu.SemaphoreType.DMA,
            pltpu.SemaphoreType.DMA,
        ],
        compiler_params=pltpu.CompilerParams(needs_layout_passes=False),
    )(h_lo, h_hi, e_lo, e_hi, src, dst)


def _edge_mlp_kernel(d2_ref, w1_ref, b1_ref, w2_ref, b2_ref, elo_ref, ehi_ref):
    d = jnp.sqrt(jnp.maximum(d2_ref[...], 0.0))
    centers = lax.broadcasted_iota(jnp.int32, (EB, RBF_K), 1).astype(jnp.float32) * CSTEP
    rbf = jnp.exp(-GAMMA * (d - centers) ** 2)
    t = jnp.dot(rbf, w1_ref[...], preferred_element_type=jnp.float32)
    t = jnp.maximum(t + b1_ref[...], 0.0)
    e = jnp.dot(t, w2_ref[...], preferred_element_type=jnp.float32) + b2_ref[...]
    elo_ref[...] = e[:, :HH]
    ehi_ref[...] = e[:, HH:]


def _tc_edge_mlp(d2, w1, b1, w2, b2):
    return pl.pallas_call(
        _edge_mlp_kernel,
        grid=(EP // EB,),
        in_specs=[
            pl.BlockSpec((EB, 1), lambda i: (i, 0)),
            pl.BlockSpec((RBF_K, H), lambda i: (0, 0)),
            pl.BlockSpec((1, H), lambda i: (0, 0)),
            pl.BlockSpec((H, H), lambda i: (0, 0)),
            pl.BlockSpec((1, H), lambda i: (0, 0)),
        ],
        out_specs=[
            pl.BlockSpec((EB, HH), lambda i: (i, 0)),
            pl.BlockSpec((EB, HH), lambda i: (i, 0)),
        ],
        out_shape=[
            jax.ShapeDtypeStruct((EP, HH), jnp.float32),
            jax.ShapeDtypeStruct((EP, HH), jnp.float32),
        ],
    )(d2, w1, b1, w2, b2)


def _h0_kernel(x_ref, w_ref, b_ref, lo_ref, hi_ref):
    h = jnp.dot(x_ref[...], w_ref[...], preferred_element_type=jnp.float32)
    h = h + b_ref[...]
    lo_ref[...] = h[:, :HH]
    hi_ref[...] = h[:, HH:]


def _tc_h0(xp, w, b):
    return pl.pallas_call(
        _h0_kernel,
        grid=(NP // NB,),
        in_specs=[
            pl.BlockSpec((NB, 16), lambda i: (i, 0)),
            pl.BlockSpec((16, H), lambda i: (0, 0)),
            pl.BlockSpec((1, H), lambda i: (0, 0)),
        ],
        out_specs=[
            pl.BlockSpec((NB, HH), lambda i: (i, 0)),
            pl.BlockSpec((NB, HH), lambda i: (i, 0)),
        ],
        out_shape=[
            jax.ShapeDtypeStruct((NP, HH), jnp.float32),
            jax.ShapeDtypeStruct((NP, HH), jnp.float32),
        ],
    )(xp, w, b)


def _node_mlp_kernel(hlo_ref, hhi_ref, alo_ref, ahi_ref,
                     w1_ref, b1_ref, w2_ref, b2_ref, olo_ref, ohi_ref):
    zlo = hlo_ref[...] + alo_ref[...]
    zhi = hhi_ref[...] + ahi_ref[...]
    w1 = w1_ref[...]
    t = jnp.dot(zlo, w1[:HH, :], preferred_element_type=jnp.float32)
    t = t + jnp.dot(zhi, w1[HH:, :], preferred_element_type=jnp.float32)
    t = jnp.maximum(t + b1_ref[...], 0.0)
    u = jnp.dot(t, w2_ref[...], preferred_element_type=jnp.float32) + b2_ref[...]
    olo_ref[...] = jnp.maximum(u[:, :HH], 0.0) + hlo_ref[...]
    ohi_ref[...] = jnp.maximum(u[:, HH:], 0.0) + hhi_ref[...]


def _tc_node_mlp(h_lo, h_hi, a_lo, a_hi, w1, b1, w2, b2):
    return pl.pallas_call(
        _node_mlp_kernel,
        grid=(NP // NB,),
        in_specs=[
            pl.BlockSpec((NB, HH), lambda i: (i, 0)),
            pl.BlockSpec((NB, HH), lambda i: (i, 0)),
            pl.BlockSpec((NB, HH), lambda i: (i, 0)),
            pl.BlockSpec((NB, HH), lambda i: (i, 0)),
            pl.BlockSpec((H, H), lambda i: (0, 0)),
            pl.BlockSpec((1, H), lambda i: (0, 0)),
            pl.BlockSpec((H, H), lambda i: (0, 0)),
            pl.BlockSpec((1, H), lambda i: (0, 0)),
        ],
        out_specs=[
            pl.BlockSpec((NB, HH), lambda i: (i, 0)),
            pl.BlockSpec((NB, HH), lambda i: (i, 0)),
        ],
        out_shape=[
            jax.ShapeDtypeStruct((NP, HH), jnp.float32),
            jax.ShapeDtypeStruct((NP, HH), jnp.float32),
        ],
    )(h_lo, h_hi, a_lo, a_hi, w1, b1, w2, b2)


def _pool_head_kernel(b_ref, hlo_ref, hhi_ref, w1_ref, b1_ref, w2_ref, b2_ref,
                      out_ref, glo, ghi, cnt):
    i = pl.program_id(0)

    @pl.when(i == 0)
    def _():
        glo[...] = jnp.zeros_like(glo)
        ghi[...] = jnp.zeros_like(ghi)
        cnt[...] = jnp.zeros_like(cnt)

    seg = b_ref[0]
    gids = lax.broadcasted_iota(jnp.int32, (B, NB), 0)
    m = (seg == gids).astype(jnp.float32)
    glo[...] += jnp.dot(m, hlo_ref[...], preferred_element_type=jnp.float32)
    ghi[...] += jnp.dot(m, hhi_ref[...], preferred_element_type=jnp.float32)
    cnt[...] += jnp.sum(m, axis=1, keepdims=True)

    @pl.when(i == pl.num_programs(0) - 1)
    def _():
        c = jnp.maximum(cnt[...], 1.0)
        gl = glo[...] / c
        gh = ghi[...] / c
        w1 = w1_ref[...]
        t = jnp.dot(gl, w1[:HH, :], preferred_element_type=jnp.float32)
        t = t + jnp.dot(gh, w1[HH:, :], preferred_element_type=jnp.float32)
        t = jnp.maximum(t + b1_ref[...], 0.0)
        out_ref[...] = (jnp.dot(t, w2_ref[...], preferred_element_type=jnp.float32)
                        + b2_ref[...])


def _tc_pool_head(batch2, h_lo, h_hi, w1, b1, w2, b2):
    return pl.pallas_call(
        _pool_head_kernel,
        grid=(NP // NB,),
        in_specs=[
            pl.BlockSpec((1, 1, NB), lambda i: (i, 0, 0)),
            pl.BlockSpec((NB, HH), lambda i: (i, 0)),
            pl.BlockSpec((NB, HH), lambda i: (i, 0)),
            pl.BlockSpec((H, HH), lambda i: (0, 0)),
            pl.BlockSpec((1, HH), lambda i: (0, 0)),
            pl.BlockSpec((HH, 1), lambda i: (0, 0)),
            pl.BlockSpec((1, 1), lambda i: (0, 0)),
        ],
        out_specs=pl.BlockSpec((B, 1), lambda i: (0, 0)),
        out_shape=jax.ShapeDtypeStruct((B, 1), jnp.float32),
        scratch_shapes=[
            pltpu.VMEM((B, HH), jnp.float32),
            pltpu.VMEM((B, HH), jnp.float32),
            pltpu.VMEM((B, 1), jnp.float32),
        ],
    )(batch2, h_lo, h_hi, w1, b1, w2, b2)


def kernel(x, pos, edge_index, batch, node_in_W, node_in_b, edge_W1, edge_b1,
           edge_W2, edge_b2, conv_W1, conv_b1, conv_W2, conv_b2,
           head_W1, head_b1, head_W2, head_b2):
    f32 = jnp.float32
    px = pos[:, 0]
    py = pos[:, 1]
    pz = pos[:, 2]
    src = edge_index[0]
    dst = edge_index[1]
    src_p = jnp.concatenate([src, jnp.zeros((EP - E,), jnp.int32)])
    dst_p = jnp.concatenate([dst, jnp.full((EP - E,), N, jnp.int32)])
    xp = jnp.zeros((NP, 16), f32).at[:N, :9].set(x)
    w16 = jnp.zeros((16, H), f32).at[:9, :].set(node_in_W)
    batch_p = jnp.concatenate([batch, jnp.full((NP - N,), B, jnp.int32)])
    batch2 = batch_p.reshape(NP // NB, 1, NB)

    d2 = _sc_dist(px, py, pz, src_p, dst_p).reshape(EP, 1)
    e_lo, e_hi = _tc_edge_mlp(d2, edge_W1, edge_b1.reshape(1, H),
                              edge_W2, edge_b2.reshape(1, H))

    h_lo, h_hi = _tc_h0(xp, w16, node_in_b.reshape(1, H))

    for i in range(L):
        a_lo, a_hi = _sc_msg(h_lo, h_hi, e_lo, e_hi, src_p, dst_p)
        h_lo, h_hi = _tc_node_mlp(h_lo, h_hi, a_lo, a_hi,
                                  conv_W1[i], conv_b1[i].reshape(1, H),
                                  conv_W2[i], conv_b2[i].reshape(1, H))

    out = _tc_pool_head(batch2, h_lo, h_hi, head_W1, head_b1.reshape(1, HH),
                        head_W2, head_b2.reshape(1, 1))
    return out[:, 0]
